# baseline (speedup 1.0000x reference)
import jax
import jax.numpy as jnp
from jax.experimental import pallas as pl
from jax.experimental.pallas import tpu as pltpu

_BLOCK_ROWS = 256


def _decode_body(xa_ref, xb_ref, cb_ref, o_ref):
    cb = cb_ref[...]
    for j, xr in enumerate((xa_ref, xb_ref)):
        x = xr[...]
        m = jnp.max(x, axis=-1, keepdims=True)
        e = jnp.exp(x - m)
        s = jnp.sum(e, axis=-1, keepdims=True)
        o_ref[j] = jnp.dot(e, cb, preferred_element_type=jnp.float32) / s


def kernel(cls_logits, codebook):
    n, k = cls_logits.shape
    k2, d = codebook.shape
    assert k == k2
    br = _BLOCK_ROWS
    nsteps = n // (2 * br)
    out = pl.pallas_call(
        _decode_body,
        grid=(nsteps,),
        in_specs=[
            pl.BlockSpec((br, k), lambda i: (i, 0)),
            pl.BlockSpec((br, k), lambda i, _n=nsteps: (i + _n, 0)),
            pl.BlockSpec((k, d), lambda i: (0, 0)),
        ],
        out_specs=pl.BlockSpec((2, br, d), lambda i: (0, i, 0)),
        out_shape=jax.ShapeDtypeStruct((2, n // 2, d), jnp.float32),
        compiler_params=pltpu.CompilerParams(
            dimension_semantics=("arbitrary",),
        ),
    )(cls_logits, cls_logits, codebook)
    return out.reshape(n, d)


# speedup vs baseline: 1.0191x; 1.0191x over previous
import jax
import jax.numpy as jnp
from jax.experimental import pallas as pl
from jax.experimental.pallas import tpu as pltpu

_BLOCK_ROWS = 512


def _decode_body(x_ref, cb_ref, o_ref):
    x = x_ref[...]
    m = jnp.max(x, axis=-1, keepdims=True)
    e = jnp.exp(x - m)
    s = jnp.sum(e, axis=-1, keepdims=True)
    eb = e.astype(jnp.bfloat16)
    acc = jnp.dot(eb, cb_ref[...], preferred_element_type=jnp.float32)
    o_ref[...] = acc / s


def kernel(cls_logits, codebook):
    n, k = cls_logits.shape
    k2, d = codebook.shape
    assert k == k2
    br = _BLOCK_ROWS
    out = pl.pallas_call(
        _decode_body,
        grid=(n // br,),
        in_specs=[
            pl.BlockSpec((br, k), lambda i: (i, 0)),
            pl.BlockSpec((k, d), lambda i: (0, 0)),
        ],
        out_specs=pl.BlockSpec((br, d), lambda i: (i, 0)),
        out_shape=jax.ShapeDtypeStruct((n, d), jnp.float32),
        compiler_params=pltpu.CompilerParams(
            dimension_semantics=("arbitrary",),
        ),
    )(cls_logits, codebook.astype(jnp.bfloat16))
    return out
